# baseline (speedup 1.0000x reference)
import functools

import jax
import jax.numpy as jnp
from jax import lax
from jax.experimental import pallas as pl
from jax.experimental.pallas import tpu as pltpu
from jax.experimental.pallas import tpu_sc as plsc

B, L, V, D, O = 4096, 50, 1000000, 64, 1

NC, NS = 2, 16
NW = NC * NS
BPW = B // NW
IPW = BPW * L
NG = BPW // 16

CH = 400
NCHUNK = V // CH
SLOTS = -(-NCHUNK // NW)
GPC = CH // 16

_mesh = plsc.VectorSubcoreMesh(core_axis_name="c", subcore_axis_name="s")


@functools.partial(
    pl.kernel,
    out_type=jax.ShapeDtypeStruct((V,), jnp.float32),
    mesh=_mesh,
    scratch_types=[
        pltpu.VMEM((D,), jnp.float32),
        pltpu.VMEM((CH, D), jnp.float32),
        pltpu.VMEM((CH, D), jnp.float32),
        pltpu.VMEM((CH,), jnp.float32),
        pltpu.SemaphoreType.DMA,
        pltpu.SemaphoreType.DMA,
    ],
    compiler_params=pltpu.CompilerParams(
        needs_layout_passes=False, use_tc_tiling_on_sc=True),
)
def _sc_matvec(table_h, w_h, t_h, w_v, g0, g1, dots_v, sem0, sem1):
    wid = lax.axis_index("s") * NC + lax.axis_index("c")

    pltpu.sync_copy(w_h, w_v)
    w0 = w_v[pl.ds(0, 16)]
    w1 = w_v[pl.ds(16, 16)]
    w2 = w_v[pl.ds(32, 16)]
    w3 = w_v[pl.ds(48, 16)]
    wvecs = (w0, w1, w2, w3)
    wd = [wvecs[d // 16][d % 16] for d in range(D)]
    lane = lax.iota(jnp.int32, 16)
    zero = jnp.zeros((16,), jnp.float32)
    bufs = (g0, g1)
    sems = (sem0, sem1)

    for par in range(2):
        k = wid + par * NW
        pltpu.async_copy(
            table_h.at[pl.ds(k * CH, CH), :], bufs[par], sems[par])

    def slot_body(i, _):
        for par in range(2):
            slot = i * 2 + par
            k = wid + slot * NW
            buf = bufs[par]
            sem = sems[par]

            @pl.when(k < NCHUNK)
            def _process(buf=buf, sem=sem, k=k, slot=slot):
                pltpu.make_async_copy(
                    table_h.at[pl.ds(k * CH, CH), :], buf, sem).wait()

                def group_body(g, _):
                    rows = g * 16 + lane
                    acc = zero
                    for d in range(D):
                        col = jnp.full((16,), d, jnp.int32)
                        v = plsc.load_gather(buf, [rows, col])
                        acc = acc + v * wd[d]
                    dots_v[pl.ds(g * 16, 16)] = acc
                    return 0

                lax.fori_loop(0, GPC, group_body, 0)
                pltpu.sync_copy(dots_v, t_h.at[pl.ds(k * CH, CH)])
                kn = k + 2 * NW

                @pl.when(kn < NCHUNK)
                def _refill(buf=buf, sem=sem, kn=kn):
                    pltpu.async_copy(
                        table_h.at[pl.ds(kn * CH, CH), :], buf, sem)
        return 0

    lax.fori_loop(0, SLOTS // 2, slot_body, 0)
    if SLOTS % 2:
        slot = SLOTS - 1
        k = wid + slot * NW

        @pl.when(k < NCHUNK)
        def _tail():
            buf = bufs[slot % 2]
            sem = sems[slot % 2]
            pltpu.make_async_copy(
                table_h.at[pl.ds(k * CH, CH), :], buf, sem).wait()

            def group_body(g, _):
                rows = g * 16 + lane
                acc = zero
                for d in range(D):
                    col = jnp.full((16,), d, jnp.int32)
                    v = plsc.load_gather(buf, [rows, col])
                    acc = acc + v * wd[d]
                dots_v[pl.ds(g * 16, 16)] = acc
                return 0

            lax.fori_loop(0, GPC, group_body, 0)
            pltpu.sync_copy(dots_v, t_h.at[pl.ds(k * CH, CH)])


@functools.partial(
    pl.kernel,
    out_type=jax.ShapeDtypeStruct((B,), jnp.float32),
    mesh=_mesh,
    scratch_types=[
        pltpu.VMEM((IPW,), jnp.int32),
        pltpu.VMEM((BPW,), jnp.int32),
        pltpu.VMEM((16,), jnp.float32),
        pltpu.VMEM((IPW,), jnp.float32),
        pltpu.VMEM((BPW,), jnp.float32),
        pltpu.SemaphoreType.DMA,
    ],
    compiler_params=pltpu.CompilerParams(
        needs_layout_passes=False, use_tc_tiling_on_sc=False),
)
def _sc_pool(t_h, xf_h, lens_h, b_h, out_h,
             x_v, lens_v, b_v, tv_v, out_v, sem):
    wid = lax.axis_index("s") * NC + lax.axis_index("c")
    bbase = wid * BPW

    pltpu.sync_copy(xf_h.at[pl.ds(bbase * L, IPW)], x_v)
    pltpu.sync_copy(lens_h.at[pl.ds(bbase, BPW)], lens_v)
    pltpu.sync_copy(b_h, b_v)
    pltpu.async_copy(t_h.at[x_v], tv_v, sem).wait()

    bvec = b_v[...]
    zero = jnp.zeros((16,), jnp.float32)
    lane = lax.iota(jnp.int32, 16)

    for g in range(NG):
        len_vec = lens_v[pl.ds(g * 16, 16)]
        base = (g * 16 + lane) * L
        acc = zero
        for j in range(L):
            v = plsc.load_gather(tv_v, [base + j])
            acc = acc + jnp.where(len_vec > j, v, zero)
        out_v[pl.ds(g * 16, 16)] = acc / len_vec.astype(jnp.float32) + bvec

    pltpu.sync_copy(out_v, out_h.at[pl.ds(bbase, BPW)])


def kernel(x, lens, table, W, b):
    xf = x.astype(jnp.int32).reshape(B * L)
    lens32 = lens.astype(jnp.int32)
    wv = W.reshape(D).astype(jnp.float32)
    bv = jnp.broadcast_to(b.astype(jnp.float32), (16,))
    t = _sc_matvec(table.astype(jnp.float32), wv)
    out = _sc_pool(t, xf, lens32, bv)
    return out.reshape(B, O)


# speedup vs baseline: 1.6346x; 1.6346x over previous
import functools

import jax
import jax.numpy as jnp
from jax import lax
from jax.experimental import pallas as pl
from jax.experimental.pallas import tpu as pltpu
from jax.experimental.pallas import tpu_sc as plsc

B, L, V, D, O = 4096, 50, 1000000, 64, 1

NC, NS = 2, 16
NW = NC * NS
BPW = B // NW
IPW = BPW * L
NG = BPW // 16

VT = 601600
VS = V - VT
MV_BLK = 32768

CH = 400
NCHUNK = VS // CH
SLOTS = -(-NCHUNK // NW)
GPC = CH // 16

assert VT % D == 0 and VS % CH == 0


def _mv_body(t_ref, wb_ref, m_ref, o_ref):
    c1 = jnp.dot(t_ref[...], wb_ref[...], preferred_element_type=jnp.float32)
    c3 = c1.reshape(MV_BLK // D, D, D) * m_ref[...][None, :, :]
    o_ref[...] = jnp.sum(c3, axis=1)


def _table_matvec(table, wb, mask):
    return pl.pallas_call(
        _mv_body,
        grid=(pl.cdiv(VT, MV_BLK),),
        in_specs=[
            pl.BlockSpec((MV_BLK, D), lambda i: (i, 0)),
            pl.BlockSpec((D, D), lambda i: (0, 0)),
            pl.BlockSpec((D, D), lambda i: (0, 0)),
        ],
        out_specs=pl.BlockSpec((MV_BLK // D, D), lambda i: (i, 0)),
        out_shape=jax.ShapeDtypeStruct((VT // D, D), jnp.float32),
    )(table, wb, mask)


_mesh = plsc.VectorSubcoreMesh(core_axis_name="c", subcore_axis_name="s")


@functools.partial(
    pl.kernel,
    out_type=jax.ShapeDtypeStruct((VS,), jnp.float32),
    mesh=_mesh,
    scratch_types=[
        pltpu.VMEM((D,), jnp.float32),
        pltpu.VMEM((CH, D), jnp.float32),
        pltpu.VMEM((CH, D), jnp.float32),
        pltpu.VMEM((CH,), jnp.float32),
        pltpu.SemaphoreType.DMA,
        pltpu.SemaphoreType.DMA,
    ],
    compiler_params=pltpu.CompilerParams(
        needs_layout_passes=False, use_tc_tiling_on_sc=True),
)
def _sc_matvec(table_h, w_h, t_h, w_v, g0, g1, dots_v, sem0, sem1):
    wid = lax.axis_index("s") * NC + lax.axis_index("c")

    pltpu.sync_copy(w_h, w_v)
    w0 = w_v[pl.ds(0, 16)]
    w1 = w_v[pl.ds(16, 16)]
    w2 = w_v[pl.ds(32, 16)]
    w3 = w_v[pl.ds(48, 16)]
    wvecs = (w0, w1, w2, w3)
    wd = [wvecs[d // 16][d % 16] for d in range(D)]
    lane = lax.iota(jnp.int32, 16)
    zero = jnp.zeros((16,), jnp.float32)
    bufs = (g0, g1)
    sems = (sem0, sem1)

    def compute_chunk(buf, k):
        def group_body(g, _):
            rows = g * 16 + lane
            accs = [zero, zero, zero, zero]
            for d in range(D):
                col = jnp.full((16,), d, jnp.int32)
                v = plsc.load_gather(buf, [rows, col])
                accs[d % 4] = accs[d % 4] + v * wd[d]
            dots_v[pl.ds(g * 16, 16)] = (
                (accs[0] + accs[1]) + (accs[2] + accs[3]))
            return 0

        lax.fori_loop(0, GPC, group_body, 0)
        pltpu.sync_copy(dots_v, t_h.at[pl.ds(k * CH, CH)])

    for par in range(2):
        k = wid + par * NW
        pltpu.async_copy(
            table_h.at[pl.ds(VT + k * CH, CH), :], bufs[par], sems[par])

    def slot_body(i, _):
        for par in range(2):
            slot = i * 2 + par
            k = wid + slot * NW

            @pl.when(k < NCHUNK)
            def _process(k=k, par=par):
                buf = bufs[par]
                sem = sems[par]
                pltpu.make_async_copy(
                    table_h.at[pl.ds(VT + k * CH, CH), :], buf, sem).wait()
                compute_chunk(buf, k)
                kn = k + 2 * NW

                @pl.when(kn < NCHUNK)
                def _refill(kn=kn, buf=buf, sem=sem):
                    pltpu.async_copy(
                        table_h.at[pl.ds(VT + kn * CH, CH), :], buf, sem)
        return 0

    lax.fori_loop(0, SLOTS // 2, slot_body, 0)
    if SLOTS % 2:
        slot = SLOTS - 1
        k = wid + slot * NW

        @pl.when(k < NCHUNK)
        def _tail():
            buf = bufs[slot % 2]
            sem = sems[slot % 2]
            pltpu.make_async_copy(
                table_h.at[pl.ds(VT + k * CH, CH), :], buf, sem).wait()
            compute_chunk(buf, k)


@functools.partial(
    pl.kernel,
    out_type=jax.ShapeDtypeStruct((B,), jnp.float32),
    mesh=_mesh,
    scratch_types=[
        pltpu.VMEM((IPW,), jnp.int32),
        pltpu.VMEM((BPW,), jnp.int32),
        pltpu.VMEM((16,), jnp.float32),
        pltpu.VMEM((IPW,), jnp.float32),
        pltpu.VMEM((BPW,), jnp.float32),
        pltpu.SemaphoreType.DMA,
    ],
    compiler_params=pltpu.CompilerParams(
        needs_layout_passes=False, use_tc_tiling_on_sc=False),
)
def _sc_pool(t_h, xf_h, lens_h, b_h, out_h,
             x_v, lens_v, b_v, tv_v, out_v, sem):
    wid = lax.axis_index("s") * NC + lax.axis_index("c")
    bbase = wid * BPW

    pltpu.sync_copy(xf_h.at[pl.ds(bbase * L, IPW)], x_v)
    pltpu.sync_copy(lens_h.at[pl.ds(bbase, BPW)], lens_v)
    pltpu.sync_copy(b_h, b_v)
    pltpu.async_copy(t_h.at[x_v], tv_v, sem).wait()

    bvec = b_v[...]
    zero = jnp.zeros((16,), jnp.float32)
    lane = lax.iota(jnp.int32, 16)

    for g in range(NG):
        len_vec = lens_v[pl.ds(g * 16, 16)]
        base = (g * 16 + lane) * L
        acc = zero
        for j in range(L):
            v = plsc.load_gather(tv_v, [base + j])
            acc = acc + jnp.where(len_vec > j, v, zero)
        out_v[pl.ds(g * 16, 16)] = acc / len_vec.astype(jnp.float32) + bvec

    pltpu.sync_copy(out_v, out_h.at[pl.ds(bbase, BPW)])


def kernel(x, lens, table, W, b):
    xf = x.astype(jnp.int32).reshape(B * L)
    lens32 = lens.astype(jnp.int32)
    wv = W.reshape(D).astype(jnp.float32)
    wb = jnp.broadcast_to(wv[:, None], (D, D))
    mask = jnp.eye(D, dtype=jnp.float32)
    bv = jnp.broadcast_to(b.astype(jnp.float32), (16,))
    tf = table.astype(jnp.float32)
    t_tc = _table_matvec(tf, wb, mask).reshape(VT)
    t_sc = _sc_matvec(tf, wv)
    t = jnp.concatenate([t_tc, t_sc])
    out = _sc_pool(t, xf, lens32, bv)
    return out.reshape(B, O)


# speedup vs baseline: 2.3989x; 1.4676x over previous
import functools

import jax
import jax.numpy as jnp
from jax import lax
from jax.experimental import pallas as pl
from jax.experimental.pallas import tpu as pltpu
from jax.experimental.pallas import tpu_sc as plsc

B, L, V, D, O = 4096, 50, 1000000, 64, 1

NC, NS = 2, 16
NW = NC * NS
BPW = B // NW
IPW = BPW * L
NG = BPW // 16

VT = V
VS = V - VT
MV_BLK = 16384

CH = 400
NCHUNK = VS // CH
SLOTS = -(-NCHUNK // NW)
GPC = CH // 16

assert VT % D == 0 and VS % CH == 0


VA = 524288
VB = V - VA
NBLK_A = VA // MV_BLK
NBLK_B = -(-VB // MV_BLK)


def _mv_body(ta_ref, tb_ref, wb_ref, m_ref, oa_ref, ob_ref):
    for t_ref, o_ref in ((ta_ref, oa_ref), (tb_ref, ob_ref)):
        c1 = jnp.dot(t_ref[...], wb_ref[...],
                     preferred_element_type=jnp.float32)
        c3 = c1.reshape(MV_BLK // D, D, D) * m_ref[...][None, :, :]
        o_ref[...] = jnp.sum(c3, axis=1)


def _table_matvec(table, wb, mask):
    return pl.pallas_call(
        _mv_body,
        grid=(NBLK_A,),
        in_specs=[
            pl.BlockSpec((MV_BLK, D), lambda i: (i, 0)),
            pl.BlockSpec(
                (MV_BLK, D),
                lambda i: (NBLK_A + jnp.minimum(i, NBLK_B - 1), 0)),
            pl.BlockSpec((D, D), lambda i: (0, 0)),
            pl.BlockSpec((D, D), lambda i: (0, 0)),
        ],
        out_specs=[
            pl.BlockSpec((MV_BLK // D, D), lambda i: (i, 0)),
            pl.BlockSpec((MV_BLK // D, D),
                         lambda i: (jnp.minimum(i, NBLK_B - 1), 0)),
        ],
        out_shape=[
            jax.ShapeDtypeStruct((VA // D, D), jnp.float32),
            jax.ShapeDtypeStruct((VB // D, D), jnp.float32),
        ],
    )(table, table, wb, mask)


_mesh = plsc.VectorSubcoreMesh(core_axis_name="c", subcore_axis_name="s")


@functools.partial(
    pl.kernel,
    out_type=jax.ShapeDtypeStruct((VS,), jnp.float32),
    mesh=_mesh,
    scratch_types=[
        pltpu.VMEM((D,), jnp.float32),
        pltpu.VMEM((CH, D), jnp.float32),
        pltpu.VMEM((CH, D), jnp.float32),
        pltpu.VMEM((CH,), jnp.float32),
        pltpu.SemaphoreType.DMA,
        pltpu.SemaphoreType.DMA,
    ],
    compiler_params=pltpu.CompilerParams(
        needs_layout_passes=False, use_tc_tiling_on_sc=True),
)
def _sc_matvec(table_h, w_h, t_h, w_v, g0, g1, dots_v, sem0, sem1):
    wid = lax.axis_index("s") * NC + lax.axis_index("c")

    pltpu.sync_copy(w_h, w_v)
    w0 = w_v[pl.ds(0, 16)]
    w1 = w_v[pl.ds(16, 16)]
    w2 = w_v[pl.ds(32, 16)]
    w3 = w_v[pl.ds(48, 16)]
    wvecs = (w0, w1, w2, w3)
    wd = [wvecs[d // 16][d % 16] for d in range(D)]
    lane = lax.iota(jnp.int32, 16)
    zero = jnp.zeros((16,), jnp.float32)
    bufs = (g0, g1)
    sems = (sem0, sem1)

    def compute_chunk(buf, k):
        def group_body(g, _):
            rows = g * 16 + lane
            accs = [zero, zero, zero, zero]
            for d in range(D):
                col = jnp.full((16,), d, jnp.int32)
                v = plsc.load_gather(buf, [rows, col])
                accs[d % 4] = accs[d % 4] + v * wd[d]
            dots_v[pl.ds(g * 16, 16)] = (
                (accs[0] + accs[1]) + (accs[2] + accs[3]))
            return 0

        lax.fori_loop(0, GPC, group_body, 0)
        pltpu.sync_copy(dots_v, t_h.at[pl.ds(k * CH, CH)])

    for par in range(2):
        k = wid + par * NW
        pltpu.async_copy(
            table_h.at[pl.ds(VT + k * CH, CH), :], bufs[par], sems[par])

    def slot_body(i, _):
        for par in range(2):
            slot = i * 2 + par
            k = wid + slot * NW

            @pl.when(k < NCHUNK)
            def _process(k=k, par=par):
                buf = bufs[par]
                sem = sems[par]
                pltpu.make_async_copy(
                    table_h.at[pl.ds(VT + k * CH, CH), :], buf, sem).wait()
                compute_chunk(buf, k)
                kn = k + 2 * NW

                @pl.when(kn < NCHUNK)
                def _refill(kn=kn, buf=buf, sem=sem):
                    pltpu.async_copy(
                        table_h.at[pl.ds(VT + kn * CH, CH), :], buf, sem)
        return 0

    lax.fori_loop(0, SLOTS // 2, slot_body, 0)
    if SLOTS % 2:
        slot = SLOTS - 1
        k = wid + slot * NW

        @pl.when(k < NCHUNK)
        def _tail():
            buf = bufs[slot % 2]
            sem = sems[slot % 2]
            pltpu.make_async_copy(
                table_h.at[pl.ds(VT + k * CH, CH), :], buf, sem).wait()
            compute_chunk(buf, k)


@functools.partial(
    pl.kernel,
    out_type=jax.ShapeDtypeStruct((B,), jnp.float32),
    mesh=_mesh,
    scratch_types=[
        pltpu.VMEM((IPW,), jnp.int32),
        pltpu.VMEM((BPW,), jnp.int32),
        pltpu.VMEM((16,), jnp.float32),
        pltpu.VMEM((IPW,), jnp.float32),
        pltpu.VMEM((BPW,), jnp.float32),
        pltpu.SemaphoreType.DMA,
    ],
    compiler_params=pltpu.CompilerParams(
        needs_layout_passes=False, use_tc_tiling_on_sc=False),
)
def _sc_pool(t_h, xf_h, lens_h, b_h, out_h,
             x_v, lens_v, b_v, tv_v, out_v, sem):
    wid = lax.axis_index("s") * NC + lax.axis_index("c")
    bbase = wid * BPW

    pltpu.sync_copy(xf_h.at[pl.ds(bbase * L, IPW)], x_v)
    pltpu.sync_copy(lens_h.at[pl.ds(bbase, BPW)], lens_v)
    pltpu.sync_copy(b_h, b_v)
    pltpu.async_copy(t_h.at[x_v], tv_v, sem).wait()

    bvec = b_v[...]
    zero = jnp.zeros((16,), jnp.float32)
    lane = lax.iota(jnp.int32, 16)

    for g in range(NG):
        len_vec = lens_v[pl.ds(g * 16, 16)]
        base = (g * 16 + lane) * L
        acc = zero
        for j in range(L):
            v = plsc.load_gather(tv_v, [base + j])
            acc = acc + jnp.where(len_vec > j, v, zero)
        out_v[pl.ds(g * 16, 16)] = acc / len_vec.astype(jnp.float32) + bvec

    pltpu.sync_copy(out_v, out_h.at[pl.ds(bbase, BPW)])


def kernel(x, lens, table, W, b):
    xf = x.astype(jnp.int32).reshape(B * L)
    lens32 = lens.astype(jnp.int32)
    wv = W.reshape(D).astype(jnp.float32)
    wb = jnp.broadcast_to(wv[:, None], (D, D))
    mask = jnp.eye(D, dtype=jnp.float32)
    bv = jnp.broadcast_to(b.astype(jnp.float32), (16,))
    tf = table.astype(jnp.float32)
    ta, tb = _table_matvec(tf, wb, mask)
    t = jnp.concatenate([ta.reshape(VA), tb.reshape(VB)])
    out = _sc_pool(t, xf, lens32, bv)
    return out.reshape(B, O)


# speedup vs baseline: 2.4308x; 1.0133x over previous
import functools

import jax
import jax.numpy as jnp
from jax import lax
from jax.experimental import pallas as pl
from jax.experimental.pallas import tpu as pltpu
from jax.experimental.pallas import tpu_sc as plsc

B, L, V, D, O = 4096, 50, 1000000, 64, 1

NC, NS = 2, 16
NW = NC * NS
BPW = B // NW
IPW = BPW * L
NG = BPW // 16

MV_BLK = 32768


def _mv_body(t_ref, wb_ref, m_ref, o_ref):
    c1 = jnp.dot(t_ref[...], wb_ref[...], preferred_element_type=jnp.float32)
    c3 = c1.reshape(MV_BLK // D, D, D) * m_ref[...][None, :, :]
    o_ref[...] = jnp.sum(c3, axis=1)


def _table_matvec(table, wb, mask):
    return pl.pallas_call(
        _mv_body,
        grid=(pl.cdiv(V, MV_BLK),),
        in_specs=[
            pl.BlockSpec((MV_BLK, D), lambda i: (i, 0)),
            pl.BlockSpec((D, D), lambda i: (0, 0)),
            pl.BlockSpec((D, D), lambda i: (0, 0)),
        ],
        out_specs=pl.BlockSpec((MV_BLK // D, D), lambda i: (i, 0)),
        out_shape=jax.ShapeDtypeStruct((V // D, D), jnp.float32),
    )(table, wb, mask)


_mesh = plsc.VectorSubcoreMesh(core_axis_name="c", subcore_axis_name="s")


@functools.partial(
    pl.kernel,
    out_type=jax.ShapeDtypeStruct((B,), jnp.float32),
    mesh=_mesh,
    scratch_types=[
        pltpu.VMEM((IPW,), jnp.int32),
        pltpu.VMEM((BPW,), jnp.int32),
        pltpu.VMEM((16,), jnp.float32),
        pltpu.VMEM((IPW,), jnp.float32),
        pltpu.VMEM((BPW,), jnp.float32),
        pltpu.SemaphoreType.DMA,
    ],
    compiler_params=pltpu.CompilerParams(
        needs_layout_passes=False, use_tc_tiling_on_sc=False),
)
def _sc_pool(t_h, xf_h, lens_h, b_h, out_h,
             x_v, lens_v, b_v, tv_v, out_v, sem):
    wid = lax.axis_index("s") * NC + lax.axis_index("c")
    bbase = wid * BPW

    pltpu.sync_copy(xf_h.at[pl.ds(bbase * L, IPW)], x_v)
    pltpu.sync_copy(lens_h.at[pl.ds(bbase, BPW)], lens_v)
    pltpu.sync_copy(b_h, b_v)
    pltpu.async_copy(t_h.at[x_v], tv_v, sem).wait()

    bvec = b_v[...]
    zero = jnp.zeros((16,), jnp.float32)
    lane = lax.iota(jnp.int32, 16)

    for g in range(NG):
        len_vec = lens_v[pl.ds(g * 16, 16)]
        base = (g * 16 + lane) * L
        acc = zero
        for j in range(L):
            v = plsc.load_gather(tv_v, [base + j])
            acc = acc + jnp.where(len_vec > j, v, zero)
        out_v[pl.ds(g * 16, 16)] = acc / len_vec.astype(jnp.float32) + bvec

    pltpu.sync_copy(out_v, out_h.at[pl.ds(bbase, BPW)])


def kernel(x, lens, table, W, b):
    xf = x.astype(jnp.int32).reshape(B * L)
    lens32 = lens.astype(jnp.int32)
    wv = W.reshape(D).astype(jnp.float32)
    wb = jnp.broadcast_to(wv[:, None], (D, D))
    mask = jnp.eye(D, dtype=jnp.float32)
    bv = jnp.broadcast_to(b.astype(jnp.float32), (16,))
    t = _table_matvec(table.astype(jnp.float32), wb, mask).reshape(V)
    out = _sc_pool(t, xf, lens32, bv)
    return out.reshape(B, O)
